# baseline (speedup 1.0000x reference)
import functools
import math

import jax
import jax.numpy as jnp
import numpy as np
from jax import lax
from jax.experimental import pallas as pl
from jax.experimental.pallas import tpu as pltpu
from jax.experimental.pallas import tpu_sc as plsc

VOCAB = 1000000
EMBED = 64
D_MODEL = 64
BATCH = 4096
SEQ = 200
MAX_SEQ_LEN = 300

NUM_CORES = 2
NUM_SUBCORES = 16
NW = NUM_CORES * NUM_SUBCORES

ROWS = BATCH * SEQ
RPW = ROWS // NW
CHUNK = 512
NSTEP = RPW // CHUNK
NBUF = 2


def _pe_table() -> np.ndarray:
    pe = np.zeros((MAX_SEQ_LEN, D_MODEL), dtype=np.float32)
    position = np.arange(0, MAX_SEQ_LEN, dtype=np.float32)[:, None]
    div_term = np.exp(
        np.arange(0, D_MODEL, 2, dtype=np.float32) * (-math.log(10000.0) / D_MODEL)
    )
    pe[:, 0::2] = np.sin(position * div_term)
    pe[:, 1::2] = np.cos(position * div_term)
    return pe[:SEQ]


_POS = _pe_table()

@functools.cache
def _make_sc_gather():
    mesh = plsc.VectorSubcoreMesh(core_axis_name="c", subcore_axis_name="s")
    return functools.partial(
        pl.kernel,
        mesh=mesh,
        compiler_params=pltpu.CompilerParams(use_tc_tiling_on_sc=False),
        out_type=jax.ShapeDtypeStruct((ROWS, EMBED), jnp.float32),
        scratch_types=[
            pltpu.VMEM((NBUF, CHUNK), jnp.int32),
            pltpu.VMEM((NBUF, CHUNK, EMBED), jnp.float32),
            pltpu.SemaphoreType.DMA,
            pltpu.SemaphoreType.DMA,
            pltpu.SemaphoreType.DMA,
            pltpu.SemaphoreType.DMA,
        ],
    )(_sc_gather_body)


def _sc_gather_body(table_hbm, idx_hbm, out_hbm, idx_v, rows_v,
                    gsem0, gsem1, osem0, osem1):
    gsem = [gsem0, gsem1]
    osem = [osem0, osem1]
    wid = lax.axis_index("s") * NUM_CORES + lax.axis_index("c")
    base0 = wid * RPW

    def fire(step, slot):
        base = base0 + step * CHUNK
        pltpu.sync_copy(idx_hbm.at[pl.ds(base, CHUNK)], idx_v.at[slot])
        pltpu.async_copy(table_hbm.at[idx_v.at[slot]], rows_v.at[slot],
                         gsem[slot])

    for b in range(NBUF):
        fire(b, b)

    def body(step, carry):
        for b in range(NBUF):
            g = step * NBUF + b
            pltpu.make_async_copy(table_hbm.at[idx_v.at[b]], rows_v.at[b],
                                  gsem[b]).wait()
            base = base0 + g * CHUNK
            pltpu.async_copy(rows_v.at[b], out_hbm.at[pl.ds(base, CHUNK)],
                             osem[b])
            nxt = g + NBUF

            @pl.when(nxt < NSTEP)
            def _refill(b=b, g=g, nxt=nxt, base=base):
                pltpu.make_async_copy(rows_v.at[b],
                                      out_hbm.at[pl.ds(base, CHUNK)],
                                      osem[b]).wait()
                fire(nxt, b)
        return carry

    lax.fori_loop(0, NSTEP // NBUF, body, 0, unroll=False)

    for b in range(NBUF):
        g = NSTEP - NBUF + b
        base = base0 + g * CHUNK
        pltpu.make_async_copy(rows_v.at[b], out_hbm.at[pl.ds(base, CHUNK)],
                              osem[b]).wait()


BB = 8
TB = BB * SEQ


def _tc_body(emb_ref, idx_ref, wt_ref, posb_ref, out_ref):
    emb = emb_ref[...]
    msk = (idx_ref[...] != 0).astype(jnp.float32)
    emb = emb * msk
    out_ref[...] = (
        jnp.dot(emb, wt_ref[...], preferred_element_type=jnp.float32)
        + posb_ref[...]
    )


def _tc_project(gathered, idx2d, wt, posb):
    return pl.pallas_call(
        _tc_body,
        grid=(BATCH // BB,),
        in_specs=[
            pl.BlockSpec((TB, EMBED), lambda i: (i, 0)),
            pl.BlockSpec((TB, 1), lambda i: (i, 0)),
            pl.BlockSpec((EMBED, D_MODEL), lambda i: (0, 0)),
            pl.BlockSpec((TB, D_MODEL), lambda i: (0, 0)),
        ],
        out_specs=pl.BlockSpec((TB, D_MODEL), lambda i: (i, 0)),
        out_shape=jax.ShapeDtypeStruct((ROWS, D_MODEL), jnp.float32),
    )(gathered, idx2d, wt, posb)


def kernel(sentence, table, W, b):
    flat_idx = sentence.reshape(ROWS)
    gathered = _make_sc_gather()(table, flat_idx)
    posb = jnp.asarray(np.tile(_POS, (BB, 1))) + b[None, :]
    out = _tc_project(gathered, flat_idx.reshape(ROWS, 1), W.T, posb)
    return out.reshape(BATCH, SEQ, D_MODEL)


# speedup vs baseline: 2.8599x; 2.8599x over previous
import functools
import math

import jax
import jax.numpy as jnp
import numpy as np
from jax import lax
from jax.experimental import pallas as pl
from jax.experimental.pallas import tpu as pltpu
from jax.experimental.pallas import tpu_sc as plsc

VOCAB = 1000000
EMBED = 64
D_MODEL = 64
BATCH = 4096
SEQ = 200
MAX_SEQ_LEN = 300

RA = 2048
SH = 2 * RA
NBLK = -(-VOCAB // SH)
PROWS = NBLK * RA
PFLAT = 2 * PROWS

NUM_CORES = 2
NUM_SUBCORES = 16
NW = NUM_CORES * NUM_SUBCORES

ROWS = BATCH * SEQ
SPW = BATCH // NW
SEQROWS = 208
OUTROWS = BATCH * SEQROWS
CHSEQ = 2
CH = CHSEQ * SEQ
NSTEP = SPW // CHSEQ
NBUF = 2


def _pe_table() -> np.ndarray:
    pe = np.zeros((MAX_SEQ_LEN, D_MODEL), dtype=np.float32)
    position = np.arange(0, MAX_SEQ_LEN, dtype=np.float32)[:, None]
    div_term = np.exp(
        np.arange(0, D_MODEL, 2, dtype=np.float32) * (-math.log(10000.0) / D_MODEL)
    )
    pe[:, 0::2] = np.sin(position * div_term)
    pe[:, 1::2] = np.cos(position * div_term)
    return pe[:SEQ]


_POS104 = np.zeros((SEQROWS // 2, 2 * D_MODEL), dtype=np.float32)
_POS104[: SEQ // 2] = _pe_table().reshape(SEQ // 2, 2 * D_MODEL)


def _proj_body(t_ref, w_ref, out_ref):
    x = t_ref[...]
    w = w_ref[...]
    dn = (((0,), (0,)), ((), ()))
    ya = lax.dot_general(x[:, :RA], w, dn, preferred_element_type=jnp.float32)
    yb = lax.dot_general(x[:, RA:], w, dn, preferred_element_type=jnp.float32)
    rid = lax.broadcasted_iota(jnp.int32, (RA, 1), 0) + pl.program_id(0) * SH
    ya = jnp.where(rid == 0, 0.0, ya)
    out_ref[...] = jnp.concatenate([ya, yb], axis=1)


def _project(table_t, wt):
    return pl.pallas_call(
        _proj_body,
        grid=(NBLK,),
        in_specs=[
            pl.BlockSpec((EMBED, SH), lambda i: (0, i)),
            pl.BlockSpec((EMBED, D_MODEL), lambda i: (0, 0)),
        ],
        out_specs=pl.BlockSpec((RA, 2 * D_MODEL), lambda i: (i, 0)),
        out_shape=jax.ShapeDtypeStruct((PROWS, 2 * D_MODEL), jnp.float32),
    )(table_t, wt)


@functools.cache
def _make_sc_gather():
    mesh = plsc.VectorSubcoreMesh(core_axis_name="c", subcore_axis_name="s")
    return functools.partial(
        pl.kernel,
        mesh=mesh,
        compiler_params=pltpu.CompilerParams(use_tc_tiling_on_sc=False),
        out_type=jax.ShapeDtypeStruct((OUTROWS, EMBED), jnp.float32),
        scratch_types=[
            pltpu.VMEM((NBUF, CH), jnp.int32),
            pltpu.VMEM((NBUF, CH, EMBED), jnp.float32),
            pltpu.SemaphoreType.DMA,
            pltpu.SemaphoreType.DMA,
            pltpu.SemaphoreType.DMA,
            pltpu.SemaphoreType.DMA,
        ],
    )(_sc_gather_body)


def _sc_gather_body(table_hbm, idx_hbm, out_hbm, idx_v, rows_v,
                    gsem0, gsem1, osem0, osem1):
    gsem = [gsem0, gsem1]
    osem = [osem0, osem1]
    wid = lax.axis_index("s") * NUM_CORES + lax.axis_index("c")
    seq0 = wid * SPW

    def fire(step, slot):
        base = (seq0 + step * CHSEQ) * SEQ
        pltpu.sync_copy(idx_hbm.at[pl.ds(base, CH)], idx_v.at[slot])
        for g in range(CH // 16):
            v = idx_v[slot, pl.ds(g * 16, 16)]
            s = jnp.bitwise_and(v, SH - 1)
            bse = v - s
            p = bse + jnp.where(s < RA, s * 2, (s - RA) * 2 + 1)
            idx_v[slot, pl.ds(g * 16, 16)] = p
        pltpu.async_copy(table_hbm.at[idx_v.at[slot]], rows_v.at[slot],
                         gsem[slot])

    def put(step, slot):
        b0 = seq0 + step * CHSEQ
        for k in range(CHSEQ):
            pltpu.async_copy(
                rows_v.at[slot, pl.ds(k * SEQ, SEQ)],
                out_hbm.at[pl.ds((b0 + k) * SEQROWS, SEQ)],
                osem[slot],
            )

    def put_wait(step, slot):
        b0 = seq0 + step * CHSEQ
        for k in range(CHSEQ):
            pltpu.make_async_copy(
                rows_v.at[slot, pl.ds(k * SEQ, SEQ)],
                out_hbm.at[pl.ds((b0 + k) * SEQROWS, SEQ)],
                osem[slot],
            ).wait()

    for b in range(NBUF):
        fire(b, b)

    def body(step, carry):
        for b in range(NBUF):
            g = step * NBUF + b
            pltpu.make_async_copy(table_hbm.at[idx_v.at[b]], rows_v.at[b],
                                  gsem[b]).wait()
            put(g, b)
            nxt = g + NBUF

            @pl.when(nxt < NSTEP)
            def _refill(b=b, g=g, nxt=nxt):
                put_wait(g, b)
                fire(nxt, b)
        return carry

    lax.fori_loop(0, NSTEP // NBUF, body, 0, unroll=False)

    for b in range(NBUF):
        put_wait(NSTEP - NBUF + b, b)


BB = 1024
RG = 8
NGR = SEQROWS // 2 // RG


def _b_body(g3_ref, eye_ref, pbt_ref, out_ref):
    eye = eye_ref[...]
    dn = (((1,), (1,)), ((), ()))
    for k in range(RG):
        x2 = g3_ref[:, k, :]
        tr = lax.dot_general(eye, x2, dn,
                             preferred_element_type=jnp.float32)
        y = tr + pbt_ref[0, :, k][:, None]
        out_ref[2 * k, :, :] = y[:D_MODEL, :]
        out_ref[2 * k + 1, :, :] = y[D_MODEL:, :]


def _pos_transpose(g3, eye, pbt):
    return pl.pallas_call(
        _b_body,
        grid=(NGR, BATCH // BB),
        in_specs=[
            pl.BlockSpec((BB, RG, 2 * D_MODEL), lambda r, c: (c, r, 0)),
            pl.BlockSpec((2 * D_MODEL, 2 * D_MODEL), lambda r, c: (0, 0)),
            pl.BlockSpec((1, 2 * D_MODEL, RG), lambda r, c: (r, 0, 0)),
        ],
        out_specs=pl.BlockSpec((2 * RG, D_MODEL, BB), lambda r, c: (r, 0, c)),
        out_shape=jax.ShapeDtypeStruct((SEQ, D_MODEL, BATCH), jnp.float32),
    )(g3, eye, pbt)


def kernel(sentence, table, W, b):
    wt = W.T
    proj = _project(table.T, wt)
    pflat = proj.reshape(PFLAT, D_MODEL)
    flat_idx = sentence.reshape(ROWS)
    g = _make_sc_gather()(pflat, flat_idx)
    g3 = g.reshape(BATCH, SEQROWS // 2, 2 * D_MODEL)
    pbt = (jnp.asarray(_POS104) + jnp.tile(b, 2)[None, :]).reshape(
        NGR, RG, 2 * D_MODEL
    ).transpose(0, 2, 1)
    eye = jnp.asarray(np.eye(2 * D_MODEL, dtype=np.float32))
    x = _pos_transpose(g3, eye, pbt)
    return jnp.transpose(x, (2, 0, 1))


# speedup vs baseline: 3.0010x; 1.0493x over previous
import functools
import math

import jax
import jax.numpy as jnp
import numpy as np
from jax import lax
from jax.experimental import pallas as pl
from jax.experimental.pallas import tpu as pltpu
from jax.experimental.pallas import tpu_sc as plsc

VOCAB = 1000000
EMBED = 64
D_MODEL = 64
BATCH = 4096
SEQ = 200
MAX_SEQ_LEN = 300

RA = 2048
SH = 2 * RA
NBLK = -(-VOCAB // SH)
PROWS = NBLK * RA
PFLAT = 2 * PROWS

NUM_CORES = 2
NUM_SUBCORES = 16
NW = NUM_CORES * NUM_SUBCORES

ROWS = BATCH * SEQ
NK = 4
BCH = BATCH // NK
SPWC = BCH // NW
SEQROWS = 208
CHOUT = BCH * SEQROWS
CHSEQ = 2
CH = CHSEQ * SEQ
NSTEPC = SPWC // CHSEQ
NBUF = 2


def _pe_table() -> np.ndarray:
    pe = np.zeros((MAX_SEQ_LEN, D_MODEL), dtype=np.float32)
    position = np.arange(0, MAX_SEQ_LEN, dtype=np.float32)[:, None]
    div_term = np.exp(
        np.arange(0, D_MODEL, 2, dtype=np.float32) * (-math.log(10000.0) / D_MODEL)
    )
    pe[:, 0::2] = np.sin(position * div_term)
    pe[:, 1::2] = np.cos(position * div_term)
    return pe[:SEQ]


_POS104 = np.zeros((SEQROWS // 2, 2 * D_MODEL), dtype=np.float32)
_POS104[: SEQ // 2] = _pe_table().reshape(SEQ // 2, 2 * D_MODEL)


def _proj_body(t_ref, w_ref, out_ref):
    x = t_ref[...]
    w = w_ref[...]
    dn = (((0,), (0,)), ((), ()))
    ya = lax.dot_general(x[:, :RA], w, dn, preferred_element_type=jnp.float32)
    yb = lax.dot_general(x[:, RA:], w, dn, preferred_element_type=jnp.float32)
    rid = lax.broadcasted_iota(jnp.int32, (RA, 1), 0) + pl.program_id(0) * SH
    ya = jnp.where(rid == 0, 0.0, ya)
    out_ref[...] = jnp.concatenate([ya, yb], axis=1)


def _project(table_t, wt):
    return pl.pallas_call(
        _proj_body,
        grid=(NBLK,),
        in_specs=[
            pl.BlockSpec((EMBED, SH), lambda i: (0, i)),
            pl.BlockSpec((EMBED, D_MODEL), lambda i: (0, 0)),
        ],
        out_specs=pl.BlockSpec((RA, 2 * D_MODEL), lambda i: (i, 0)),
        out_shape=jax.ShapeDtypeStruct((PROWS, 2 * D_MODEL), jnp.float32),
    )(table_t, wt)


@functools.cache
def _make_sc_gather(chunk):
    mesh = plsc.VectorSubcoreMesh(core_axis_name="c", subcore_axis_name="s")
    return functools.partial(
        pl.kernel,
        mesh=mesh,
        compiler_params=pltpu.CompilerParams(use_tc_tiling_on_sc=False),
        out_type=jax.ShapeDtypeStruct((CHOUT, EMBED), jnp.float32),
        scratch_types=[
            pltpu.VMEM((NBUF, CH), jnp.int32),
            pltpu.VMEM((NBUF, CH, EMBED), jnp.float32),
            pltpu.SemaphoreType.DMA,
            pltpu.SemaphoreType.DMA,
            pltpu.SemaphoreType.DMA,
            pltpu.SemaphoreType.DMA,
        ],
    )(functools.partial(_sc_gather_body, chunk))


def _sc_gather_body(chunk, table_hbm, idx_hbm, out_hbm, idx_v, rows_v,
                    gsem0, gsem1, osem0, osem1):
    gsem = [gsem0, gsem1]
    osem = [osem0, osem1]
    wid = lax.axis_index("s") * NUM_CORES + lax.axis_index("c")
    seq_l = wid * SPWC

    def fire(step, slot):
        base = (chunk * BCH + seq_l + step * CHSEQ) * SEQ
        pltpu.sync_copy(idx_hbm.at[pl.ds(base, CH)], idx_v.at[slot])
        for g in range(CH // 16):
            v = idx_v[slot, pl.ds(g * 16, 16)]
            s = jnp.bitwise_and(v, SH - 1)
            bse = v - s
            p = bse + jnp.where(s < RA, s * 2, (s - RA) * 2 + 1)
            idx_v[slot, pl.ds(g * 16, 16)] = p
        pltpu.async_copy(table_hbm.at[idx_v.at[slot]], rows_v.at[slot],
                         gsem[slot])

    def put(step, slot):
        b0 = seq_l + step * CHSEQ
        for k in range(CHSEQ):
            pltpu.async_copy(
                rows_v.at[slot, pl.ds(k * SEQ, SEQ)],
                out_hbm.at[pl.ds((b0 + k) * SEQROWS, SEQ)],
                osem[slot],
            )

    def put_wait(step, slot):
        b0 = seq_l + step * CHSEQ
        for k in range(CHSEQ):
            pltpu.make_async_copy(
                rows_v.at[slot, pl.ds(k * SEQ, SEQ)],
                out_hbm.at[pl.ds((b0 + k) * SEQROWS, SEQ)],
                osem[slot],
            ).wait()

    for b in range(NBUF):
        fire(b, b)

    def body(step, carry):
        for b in range(NBUF):
            g = step * NBUF + b
            pltpu.make_async_copy(table_hbm.at[idx_v.at[b]], rows_v.at[b],
                                  gsem[b]).wait()
            put(g, b)
            nxt = g + NBUF

            @pl.when(nxt < NSTEPC)
            def _refill(b=b, g=g, nxt=nxt):
                put_wait(g, b)
                fire(nxt, b)
        return carry

    lax.fori_loop(0, NSTEPC // NBUF, body, 0, unroll=False)

    for b in range(NBUF):
        put_wait(NSTEPC - NBUF + b, b)


RG = 8
NGR = SEQROWS // 2 // RG


def _b_body(g3_ref, eye_ref, pbt_ref, out_ref):
    eye = eye_ref[...]
    dn = (((1,), (1,)), ((), ()))
    for k in range(RG):
        x2 = g3_ref[:, k, :]
        tr = lax.dot_general(eye, x2, dn,
                             preferred_element_type=jnp.float32)
        y = tr + pbt_ref[0, :, k][:, None]
        out_ref[2 * k, :, :] = y[:D_MODEL, :]
        out_ref[2 * k + 1, :, :] = y[D_MODEL:, :]


def _b_body_acc(g3_ref, eye_ref, pbt_ref, xin_ref, out_ref):
    _b_body(g3_ref, eye_ref, pbt_ref, out_ref)


@functools.cache
def _make_pos_transpose(chunk, first):
    in_specs = [
        pl.BlockSpec((BCH, RG, 2 * D_MODEL), lambda r: (0, r, 0)),
        pl.BlockSpec((2 * D_MODEL, 2 * D_MODEL), lambda r: (0, 0)),
        pl.BlockSpec((1, 2 * D_MODEL, RG), lambda r: (r, 0, 0)),
    ]
    kwargs = {}
    if not first:
        in_specs.append(pl.BlockSpec(memory_space=pl.ANY))
        kwargs["input_output_aliases"] = {3: 0}
    return pl.pallas_call(
        _b_body if first else _b_body_acc,
        grid=(NGR,),
        in_specs=in_specs,
        out_specs=pl.BlockSpec((2 * RG, D_MODEL, BCH),
                               lambda r, c=chunk: (r, 0, c)),
        out_shape=jax.ShapeDtypeStruct((SEQ, D_MODEL, BATCH), jnp.float32),
        **kwargs,
    )


def kernel(sentence, table, W, b):
    wt = W.T
    proj = _project(table.T, wt)
    pflat = proj.reshape(PFLAT, D_MODEL)
    flat_idx = sentence.reshape(ROWS)
    pbt = (jnp.asarray(_POS104) + jnp.tile(b, 2)[None, :]).reshape(
        NGR, RG, 2 * D_MODEL
    ).transpose(0, 2, 1)
    eye = jnp.asarray(np.eye(2 * D_MODEL, dtype=np.float32))
    x = None
    for k in range(NK):
        gk = _make_sc_gather(k)(pflat, flat_idx)
        g3k = gk.reshape(BCH, SEQROWS // 2, 2 * D_MODEL)
        if x is None:
            x = _make_pos_transpose(k, True)(g3k, eye, pbt)
        else:
            x = _make_pos_transpose(k, False)(g3k, eye, pbt, x)
    return jnp.transpose(x, (2, 0, 1))


# speedup vs baseline: 3.0897x; 1.0296x over previous
import functools
import math

import jax
import jax.numpy as jnp
import numpy as np
from jax import lax
from jax.experimental import pallas as pl
from jax.experimental.pallas import tpu as pltpu
from jax.experimental.pallas import tpu_sc as plsc

VOCAB = 1000000
EMBED = 64
D_MODEL = 64
BATCH = 4096
SEQ = 200
MAX_SEQ_LEN = 300

HD = D_MODEL // 2
QA = 1024
NQ = 4
SH = NQ * QA
NBLK = -(-VOCAB // SH)
PROWS = NBLK * QA
PFLAT = NQ * PROWS

NUM_CORES = 2
NUM_SUBCORES = 16
NW = NUM_CORES * NUM_SUBCORES

ROWS = BATCH * SEQ
NK = 4
BCH = BATCH // NK
SPWC = BCH // NW
SEQROWS = 224
CHOUT = BCH * SEQROWS
CHSEQ = 2
CH = CHSEQ * SEQ
NSTEPC = SPWC // CHSEQ
NBUF = 2


def _pe_table() -> np.ndarray:
    pe = np.zeros((MAX_SEQ_LEN, D_MODEL), dtype=np.float32)
    position = np.arange(0, MAX_SEQ_LEN, dtype=np.float32)[:, None]
    div_term = np.exp(
        np.arange(0, D_MODEL, 2, dtype=np.float32) * (-math.log(10000.0) / D_MODEL)
    )
    pe[:, 0::2] = np.sin(position * div_term)
    pe[:, 1::2] = np.cos(position * div_term)
    return pe[:SEQ]


_POS224 = np.zeros((SEQROWS, D_MODEL), dtype=np.float32)
_POS224[:SEQ] = _pe_table()


def _pack_bf16_pair(lo, hi):
    lo16 = lax.bitcast_convert_type(lo.astype(jnp.bfloat16), jnp.uint16)
    hi16 = lax.bitcast_convert_type(hi.astype(jnp.bfloat16), jnp.uint16)
    word = lo16.astype(jnp.uint32) | (hi16.astype(jnp.uint32) << 16)
    return lax.bitcast_convert_type(word, jnp.float32)


def _proj_body(t_ref, wlo_ref, whi_ref, out_ref):
    x = t_ref[...]
    wlo = wlo_ref[...]
    whi = whi_ref[...]
    dn = (((0,), (0,)), ((), ()))
    words = []
    for q in range(NQ):
        xq = x[:, q * QA:(q + 1) * QA]
        ylo = lax.dot_general(xq, wlo, dn, preferred_element_type=jnp.float32)
        yhi = lax.dot_general(xq, whi, dn, preferred_element_type=jnp.float32)
        if q == 0:
            rid = (lax.broadcasted_iota(jnp.int32, (QA, 1), 0)
                   + pl.program_id(0) * SH)
            ylo = jnp.where(rid == 0, 0.0, ylo)
            yhi = jnp.where(rid == 0, 0.0, yhi)
        words.append(_pack_bf16_pair(ylo, yhi))
    out_ref[...] = jnp.concatenate(words, axis=1)


def _project(table_t, wlo, whi):
    return pl.pallas_call(
        _proj_body,
        grid=(NBLK,),
        in_specs=[
            pl.BlockSpec((EMBED, SH), lambda i: (0, i)),
            pl.BlockSpec((EMBED, HD), lambda i: (0, 0)),
            pl.BlockSpec((EMBED, HD), lambda i: (0, 0)),
        ],
        out_specs=pl.BlockSpec((QA, NQ * HD), lambda i: (i, 0)),
        out_shape=jax.ShapeDtypeStruct((PROWS, NQ * HD), jnp.float32),
    )(table_t, wlo, whi)


@functools.cache
def _make_sc_gather(chunk):
    mesh = plsc.VectorSubcoreMesh(core_axis_name="c", subcore_axis_name="s")
    return functools.partial(
        pl.kernel,
        mesh=mesh,
        compiler_params=pltpu.CompilerParams(use_tc_tiling_on_sc=False),
        out_type=jax.ShapeDtypeStruct((CHOUT, HD), jnp.float32),
        scratch_types=[
            pltpu.VMEM((NBUF, CH), jnp.int32),
            pltpu.VMEM((NBUF, CH, HD), jnp.float32),
            pltpu.SemaphoreType.DMA,
            pltpu.SemaphoreType.DMA,
            pltpu.SemaphoreType.DMA,
            pltpu.SemaphoreType.DMA,
        ],
    )(functools.partial(_sc_gather_body, chunk))


def _sc_gather_body(chunk, table_hbm, idx_hbm, out_hbm, idx_v, rows_v,
                    gsem0, gsem1, osem0, osem1):
    gsem = [gsem0, gsem1]
    osem = [osem0, osem1]
    wid = lax.axis_index("s") * NUM_CORES + lax.axis_index("c")
    seq_l = wid * SPWC

    def fire(step, slot):
        base = (chunk * BCH + seq_l + step * CHSEQ) * SEQ
        pltpu.sync_copy(idx_hbm.at[pl.ds(base, CH)], idx_v.at[slot])
        for g in range(CH // 16):
            v = idx_v[slot, pl.ds(g * 16, 16)]
            p = ((v & ~(SH - 1))
                 + ((v & (QA - 1)) << 2)
                 + ((v >> 10) & (NQ - 1)))
            idx_v[slot, pl.ds(g * 16, 16)] = p
        pltpu.async_copy(table_hbm.at[idx_v.at[slot]], rows_v.at[slot],
                         gsem[slot])

    def put(step, slot):
        b0 = seq_l + step * CHSEQ
        for k in range(CHSEQ):
            pltpu.async_copy(
                rows_v.at[slot, pl.ds(k * SEQ, SEQ)],
                out_hbm.at[pl.ds((b0 + k) * SEQROWS, SEQ)],
                osem[slot],
            )

    def put_wait(step, slot):
        b0 = seq_l + step * CHSEQ
        for k in range(CHSEQ):
            pltpu.make_async_copy(
                rows_v.at[slot, pl.ds(k * SEQ, SEQ)],
                out_hbm.at[pl.ds((b0 + k) * SEQROWS, SEQ)],
                osem[slot],
            ).wait()

    for b in range(NBUF):
        fire(b, b)

    def body(step, carry):
        for b in range(NBUF):
            g = step * NBUF + b
            pltpu.make_async_copy(table_hbm.at[idx_v.at[b]], rows_v.at[b],
                                  gsem[b]).wait()
            put(g, b)
            nxt = g + NBUF

            @pl.when(nxt < NSTEPC)
            def _refill(b=b, g=g, nxt=nxt):
                put_wait(g, b)
                fire(nxt, b)
        return carry

    lax.fori_loop(0, NSTEPC // NBUF, body, 0, unroll=False)

    for b in range(NBUF):
        put_wait(NSTEPC - NBUF + b, b)


RG = 8
SB = 4 * RG
NGR = SEQROWS * 4 // SB // 4


def _b_body(g3_ref, eye_ref, pbt_ref, out_ref):
    eye = eye_ref[...]
    dn = (((1,), (1,)), ((), ()))
    mask_lo = jnp.uint32(0xFFFF)
    for k in range(RG):
        xw = lax.bitcast_convert_type(g3_ref[:, k, :], jnp.uint32)
        lo = lax.bitcast_convert_type((xw & mask_lo) << 16, jnp.float32)
        hi = lax.bitcast_convert_type(xw & ~mask_lo, jnp.float32)
        tr_lo = lax.dot_general(eye, lo, dn,
                                preferred_element_type=jnp.float32)
        tr_hi = lax.dot_general(eye, hi, dn,
                                preferred_element_type=jnp.float32)
        for t in range(4):
            j = 4 * k + t
            pb = pbt_ref[0, :, j][:, None]
            out_ref[j, :HD, :] = tr_lo[t * HD:(t + 1) * HD, :] + pb[:HD]
            out_ref[j, HD:, :] = tr_hi[t * HD:(t + 1) * HD, :] + pb[HD:]


def _b_body_acc(g3_ref, eye_ref, pbt_ref, xin_ref, out_ref):
    _b_body(g3_ref, eye_ref, pbt_ref, out_ref)


@functools.cache
def _make_pos_transpose(chunk, first):
    in_specs = [
        pl.BlockSpec((BCH, RG, 2 * D_MODEL), lambda r: (0, r, 0)),
        pl.BlockSpec((2 * D_MODEL, 2 * D_MODEL), lambda r: (0, 0)),
        pl.BlockSpec((1, D_MODEL, SB), lambda r: (r, 0, 0)),
    ]
    kwargs = {}
    if not first:
        in_specs.append(pl.BlockSpec(memory_space=pl.ANY))
        kwargs["input_output_aliases"] = {3: 0}
    return pl.pallas_call(
        _b_body if first else _b_body_acc,
        grid=(NGR,),
        in_specs=in_specs,
        out_specs=pl.BlockSpec((SB, D_MODEL, BCH),
                               lambda r, c=chunk: (r, 0, c)),
        out_shape=jax.ShapeDtypeStruct((SEQ, D_MODEL, BATCH), jnp.float32),
        **kwargs,
    )


def kernel(sentence, table, W, b):
    wt = W.T
    proj = _project(table.T, wt[:, :HD], wt[:, HD:])
    pflat = proj.reshape(PFLAT, HD)
    flat_idx = sentence.reshape(ROWS)
    pbt = (jnp.asarray(_POS224) + b[None, :]).reshape(
        NGR, SB, D_MODEL
    ).transpose(0, 2, 1)
    eye = jnp.asarray(np.eye(2 * D_MODEL, dtype=np.float32))
    x = None
    for k in range(NK):
        gk = _make_sc_gather(k)(pflat, flat_idx)
        g3k = gk.reshape(BCH, SEQROWS // 4, 2 * D_MODEL)
        if x is None:
            x = _make_pos_transpose(k, True)(g3k, eye, pbt)
        else:
            x = _make_pos_transpose(k, False)(g3k, eye, pbt, x)
    return jnp.transpose(x, (2, 0, 1))


# speedup vs baseline: 3.0913x; 1.0005x over previous
import functools
import math

import jax
import jax.numpy as jnp
import numpy as np
from jax import lax
from jax.experimental import pallas as pl
from jax.experimental.pallas import tpu as pltpu
from jax.experimental.pallas import tpu_sc as plsc

VOCAB = 1000000
EMBED = 64
D_MODEL = 64
BATCH = 4096
SEQ = 200
MAX_SEQ_LEN = 300

HD = D_MODEL // 2
QA = 1024
NQ = 4
SH = NQ * QA
NBLK = -(-VOCAB // SH)
PROWS = NBLK * QA
PFLAT = NQ * PROWS

NUM_CORES = 2
NUM_SUBCORES = 16
NW = NUM_CORES * NUM_SUBCORES

ROWS = BATCH * SEQ
NK = 8
BCH = BATCH // NK
SPWC = BCH // NW
SEQROWS = 224
CHOUT = BCH * SEQROWS
CHSEQ = 2
CH = CHSEQ * SEQ
NSTEPC = SPWC // CHSEQ
NBUF = 2


def _pe_table() -> np.ndarray:
    pe = np.zeros((MAX_SEQ_LEN, D_MODEL), dtype=np.float32)
    position = np.arange(0, MAX_SEQ_LEN, dtype=np.float32)[:, None]
    div_term = np.exp(
        np.arange(0, D_MODEL, 2, dtype=np.float32) * (-math.log(10000.0) / D_MODEL)
    )
    pe[:, 0::2] = np.sin(position * div_term)
    pe[:, 1::2] = np.cos(position * div_term)
    return pe[:SEQ]


_POS224 = np.zeros((SEQROWS, D_MODEL), dtype=np.float32)
_POS224[:SEQ] = _pe_table()


def _pack_bf16_pair(lo, hi):
    lo16 = lax.bitcast_convert_type(lo.astype(jnp.bfloat16), jnp.uint16)
    hi16 = lax.bitcast_convert_type(hi.astype(jnp.bfloat16), jnp.uint16)
    word = lo16.astype(jnp.uint32) | (hi16.astype(jnp.uint32) << 16)
    return lax.bitcast_convert_type(word, jnp.float32)


def _proj_body(t_ref, wlo_ref, whi_ref, out_ref):
    x = t_ref[...]
    wlo = wlo_ref[...]
    whi = whi_ref[...]
    dn = (((0,), (0,)), ((), ()))
    words = []
    for q in range(NQ):
        xq = x[:, q * QA:(q + 1) * QA]
        ylo = lax.dot_general(xq, wlo, dn, preferred_element_type=jnp.float32)
        yhi = lax.dot_general(xq, whi, dn, preferred_element_type=jnp.float32)
        if q == 0:
            rid = (lax.broadcasted_iota(jnp.int32, (QA, 1), 0)
                   + pl.program_id(0) * SH)
            ylo = jnp.where(rid == 0, 0.0, ylo)
            yhi = jnp.where(rid == 0, 0.0, yhi)
        words.append(_pack_bf16_pair(ylo, yhi))
    out_ref[...] = jnp.concatenate(words, axis=1)


def _project(table_t, wlo, whi):
    return pl.pallas_call(
        _proj_body,
        grid=(NBLK,),
        in_specs=[
            pl.BlockSpec((EMBED, SH), lambda i: (0, i)),
            pl.BlockSpec((EMBED, HD), lambda i: (0, 0)),
            pl.BlockSpec((EMBED, HD), lambda i: (0, 0)),
        ],
        out_specs=pl.BlockSpec((QA, NQ * HD), lambda i: (i, 0)),
        out_shape=jax.ShapeDtypeStruct((PROWS, NQ * HD), jnp.float32),
    )(table_t, wlo, whi)


@functools.cache
def _make_sc_gather(chunk):
    mesh = plsc.VectorSubcoreMesh(core_axis_name="c", subcore_axis_name="s")
    return functools.partial(
        pl.kernel,
        mesh=mesh,
        compiler_params=pltpu.CompilerParams(use_tc_tiling_on_sc=False),
        out_type=jax.ShapeDtypeStruct((CHOUT, HD), jnp.float32),
        scratch_types=[
            pltpu.VMEM((NBUF, CH), jnp.int32),
            pltpu.VMEM((NBUF, CH, HD), jnp.float32),
            pltpu.SemaphoreType.DMA,
            pltpu.SemaphoreType.DMA,
            pltpu.SemaphoreType.DMA,
            pltpu.SemaphoreType.DMA,
        ],
    )(functools.partial(_sc_gather_body, chunk))


def _sc_gather_body(chunk, table_hbm, idx_hbm, out_hbm, idx_v, rows_v,
                    gsem0, gsem1, osem0, osem1):
    gsem = [gsem0, gsem1]
    osem = [osem0, osem1]
    wid = lax.axis_index("s") * NUM_CORES + lax.axis_index("c")
    seq_l = wid * SPWC

    def fire(step, slot):
        base = (chunk * BCH + seq_l + step * CHSEQ) * SEQ
        pltpu.sync_copy(idx_hbm.at[pl.ds(base, CH)], idx_v.at[slot])
        for g in range(CH // 16):
            v = idx_v[slot, pl.ds(g * 16, 16)]
            p = ((v & ~(SH - 1))
                 + ((v & (QA - 1)) << 2)
                 + ((v >> 10) & (NQ - 1)))
            idx_v[slot, pl.ds(g * 16, 16)] = p
        pltpu.async_copy(table_hbm.at[idx_v.at[slot]], rows_v.at[slot],
                         gsem[slot])

    def put(step, slot):
        b0 = seq_l + step * CHSEQ
        for k in range(CHSEQ):
            pltpu.async_copy(
                rows_v.at[slot, pl.ds(k * SEQ, SEQ)],
                out_hbm.at[pl.ds((b0 + k) * SEQROWS, SEQ)],
                osem[slot],
            )

    def put_wait(step, slot):
        b0 = seq_l + step * CHSEQ
        for k in range(CHSEQ):
            pltpu.make_async_copy(
                rows_v.at[slot, pl.ds(k * SEQ, SEQ)],
                out_hbm.at[pl.ds((b0 + k) * SEQROWS, SEQ)],
                osem[slot],
            ).wait()

    for b in range(NBUF):
        fire(b, b)

    def body(step, carry):
        for b in range(NBUF):
            g = step * NBUF + b
            pltpu.make_async_copy(table_hbm.at[idx_v.at[b]], rows_v.at[b],
                                  gsem[b]).wait()
            put(g, b)
            nxt = g + NBUF

            @pl.when(nxt < NSTEPC)
            def _refill(b=b, g=g, nxt=nxt):
                put_wait(g, b)
                fire(nxt, b)
        return carry

    lax.fori_loop(0, NSTEPC // NBUF, body, 0, unroll=False)

    for b in range(NBUF):
        put_wait(NSTEPC - NBUF + b, b)


RG = 8
SB = 4 * RG
NGR = SEQROWS * 4 // SB // 4


def _b_body(g3_ref, eye_ref, pbt_ref, out_ref):
    eye = eye_ref[...]
    dn = (((1,), (1,)), ((), ()))
    mask_lo = jnp.uint32(0xFFFF)
    for k in range(RG):
        xw = lax.bitcast_convert_type(g3_ref[:, k, :], jnp.uint32)
        lo = lax.bitcast_convert_type((xw & mask_lo) << 16, jnp.float32)
        hi = lax.bitcast_convert_type(xw & ~mask_lo, jnp.float32)
        tr_lo = lax.dot_general(eye, lo, dn,
                                preferred_element_type=jnp.float32)
        tr_hi = lax.dot_general(eye, hi, dn,
                                preferred_element_type=jnp.float32)
        for t in range(4):
            j = 4 * k + t
            pb = pbt_ref[0, :, j][:, None]
            out_ref[j, :HD, :] = tr_lo[t * HD:(t + 1) * HD, :] + pb[:HD]
            out_ref[j, HD:, :] = tr_hi[t * HD:(t + 1) * HD, :] + pb[HD:]


def _b_body_acc(g3_ref, eye_ref, pbt_ref, xin_ref, out_ref):
    _b_body(g3_ref, eye_ref, pbt_ref, out_ref)


@functools.cache
def _make_pos_transpose(chunk, first):
    in_specs = [
        pl.BlockSpec((BCH, RG, 2 * D_MODEL), lambda r: (0, r, 0)),
        pl.BlockSpec((2 * D_MODEL, 2 * D_MODEL), lambda r: (0, 0)),
        pl.BlockSpec((1, D_MODEL, SB), lambda r: (r, 0, 0)),
    ]
    kwargs = {}
    if not first:
        in_specs.append(pl.BlockSpec(memory_space=pl.ANY))
        kwargs["input_output_aliases"] = {3: 0}
    return pl.pallas_call(
        _b_body if first else _b_body_acc,
        grid=(NGR,),
        in_specs=in_specs,
        out_specs=pl.BlockSpec((SB, D_MODEL, BCH),
                               lambda r, c=chunk: (r, 0, c)),
        out_shape=jax.ShapeDtypeStruct((SEQ, D_MODEL, BATCH), jnp.float32),
        **kwargs,
    )


def kernel(sentence, table, W, b):
    wt = W.T
    proj = _project(table.T, wt[:, :HD], wt[:, HD:])
    pflat = proj.reshape(PFLAT, HD)
    flat_idx = sentence.reshape(ROWS)
    pbt = (jnp.asarray(_POS224) + b[None, :]).reshape(
        NGR, SB, D_MODEL
    ).transpose(0, 2, 1)
    eye = jnp.asarray(np.eye(2 * D_MODEL, dtype=np.float32))
    x = None
    for k in range(NK):
        gk = _make_sc_gather(k)(pflat, flat_idx)
        g3k = gk.reshape(BCH, SEQROWS // 4, 2 * D_MODEL)
        if x is None:
            x = _make_pos_transpose(k, True)(g3k, eye, pbt)
        else:
            x = _make_pos_transpose(k, False)(g3k, eye, pbt, x)
    return jnp.transpose(x, (2, 0, 1))
